# baseline (speedup 1.0000x reference)
import functools

import jax
import jax.numpy as jnp
from jax import lax
from jax.experimental import pallas as pl
from jax.experimental.pallas import tpu as pltpu
from jax.experimental.pallas import tpu_sc as plsc

D = 64
DP = 128
T = 4096
TH = T // 2
B = 4
V = 256
ROWS = B * T
HALF = ROWS // 2
NC, NS, L = 2, 16, 16
NW = NC * NS
PPW = HALF // NW


_MESH = plsc.VectorSubcoreMesh(core_axis_name="c", subcore_axis_name="s")


@functools.partial(
    pl.kernel,
    out_type=jax.ShapeDtypeStruct((HALF, DP), jnp.float32),
    mesh=_MESH,
    compiler_params=pltpu.CompilerParams(use_tc_tiling_on_sc=False),
    scratch_types=[
        pltpu.VMEM((PPW,), jnp.int32),
        pltpu.VMEM((PPW,), jnp.int32),
        pltpu.VMEM((PPW, D), jnp.float32),
        pltpu.VMEM((PPW, D), jnp.float32),
        pltpu.SemaphoreType.DMA,
        pltpu.SemaphoreType.DMA,
        pltpu.SemaphoreType.DMA,
        pltpu.SemaphoreType.DMA,
    ],
)
def _sc_gather(x_hbm, table_hbm, out_hbm, idxa_v, idxb_v, rowsa_v, rowsb_v,
               sema, semb, wsema, wsemb):
    wid = lax.axis_index("s") * NC + lax.axis_index("c")
    base = wid * PPW
    bb = wid // 8
    flata = bb * T + (wid % 8) * PPW
    pltpu.sync_copy(x_hbm.at[pl.ds(flata, PPW)], idxa_v)
    ga = pltpu.async_copy(table_hbm.at[idxa_v], rowsa_v, sema)
    pltpu.sync_copy(x_hbm.at[pl.ds(flata + TH, PPW)], idxb_v)
    gb = pltpu.async_copy(table_hbm.at[idxb_v], rowsb_v, semb)
    out_slice = out_hbm.at[pl.ds(base, PPW)]
    ga.wait()
    wa = pltpu.async_copy(rowsa_v, out_slice.at[:, pl.ds(0, D)], wsema)
    gb.wait()
    wb = pltpu.async_copy(rowsb_v, out_slice.at[:, pl.ds(D, D)], wsemb)
    wa.wait()
    wb.wait()


def _finish_body(g_ref, pos_ref, w_ref, b_ref, y_ref):
    dn = (((1,), (1,)), ((), ()))
    w = w_ref[...]
    bias = b_ref[...]
    y_ref[0, :TH] = lax.dot_general(g_ref[:, :D] + pos_ref[:TH], w, dn,
                                    preferred_element_type=jnp.float32) + bias
    y_ref[0, TH:] = lax.dot_general(g_ref[:, D:] + pos_ref[TH:], w, dn,
                                    preferred_element_type=jnp.float32) + bias


def _tc_finish(g2, pos_embed, W, b2d):
    return pl.pallas_call(
        _finish_body,
        grid=(B,),
        in_specs=[
            pl.BlockSpec((TH, DP), lambda bb: (bb, 0)),
            pl.BlockSpec((T, D), lambda bb: (0, 0)),
            pl.BlockSpec((D, D), lambda bb: (0, 0)),
            pl.BlockSpec((1, D), lambda bb: (0, 0)),
        ],
        out_specs=pl.BlockSpec((1, T, D), lambda bb: (bb, 0, 0)),
        out_shape=jax.ShapeDtypeStruct((B, T, D), jnp.float32),
    )(g2, pos_embed, W, b2d)


def kernel(x, byte_embed, pos_embed, W, b):
    x_flat = x.reshape(ROWS).astype(jnp.int32)
    g2 = _sc_gather(x_flat, byte_embed)
    return _tc_finish(g2, pos_embed, W, b.reshape(1, D))


# speedup vs baseline: 1.0022x; 1.0022x over previous
import functools

import jax
import jax.numpy as jnp
from jax import lax
from jax.experimental import pallas as pl
from jax.experimental.pallas import tpu as pltpu
from jax.experimental.pallas import tpu_sc as plsc

D = 64
DP = 128
T = 4096
TH = T // 2
B = 4
V = 256
ROWS = B * T
HALF = ROWS // 2
NC, NS, L = 2, 16, 16
NW = NC * NS
PPW = HALF // NW


_MESH = plsc.VectorSubcoreMesh(core_axis_name="c", subcore_axis_name="s")


@functools.partial(
    pl.kernel,
    out_type=jax.ShapeDtypeStruct((HALF, DP), jnp.float32),
    mesh=_MESH,
    compiler_params=pltpu.CompilerParams(use_tc_tiling_on_sc=False),
    scratch_types=[
        pltpu.VMEM((PPW,), jnp.int32),
        pltpu.VMEM((PPW,), jnp.int32),
        pltpu.VMEM((PPW, D), jnp.float32),
        pltpu.VMEM((PPW, D), jnp.float32),
        pltpu.SemaphoreType.DMA,
        pltpu.SemaphoreType.DMA,
    ],
)
def _sc_gather(x_hbm, table_hbm, out_hbm, idxa_v, idxb_v, rowsa_v, rowsb_v,
               sema, semb):
    wid = lax.axis_index("s") * NC + lax.axis_index("c")
    base = wid * PPW
    bb = wid // 8
    flata = bb * T + (wid % 8) * PPW
    pltpu.sync_copy(x_hbm.at[pl.ds(flata, PPW)], idxa_v)
    ga = pltpu.async_copy(table_hbm.at[idxa_v], rowsa_v, sema)
    pltpu.sync_copy(x_hbm.at[pl.ds(flata + TH, PPW)], idxb_v)
    gb = pltpu.async_copy(table_hbm.at[idxb_v], rowsb_v, semb)
    out_slice = out_hbm.at[pl.ds(base, PPW)]
    ga.wait()
    pltpu.sync_copy(rowsa_v, out_slice.at[:, pl.ds(0, D)])
    gb.wait()
    pltpu.sync_copy(rowsb_v, out_slice.at[:, pl.ds(D, D)])


def _finish_body(g_ref, pos_ref, w_ref, b_ref, y_ref):
    dn = (((1,), (1,)), ((), ()))
    w = w_ref[...]
    bias = b_ref[...]
    y_ref[0, :TH] = lax.dot_general(g_ref[:, :D] + pos_ref[:TH], w, dn,
                                    preferred_element_type=jnp.float32) + bias
    y_ref[0, TH:] = lax.dot_general(g_ref[:, D:] + pos_ref[TH:], w, dn,
                                    preferred_element_type=jnp.float32) + bias


def _tc_finish(g2, pos_embed, W, b2d):
    return pl.pallas_call(
        _finish_body,
        grid=(B,),
        in_specs=[
            pl.BlockSpec((TH, DP), lambda bb: (bb, 0)),
            pl.BlockSpec((T, D), lambda bb: (0, 0)),
            pl.BlockSpec((D, D), lambda bb: (0, 0)),
            pl.BlockSpec((1, D), lambda bb: (0, 0)),
        ],
        out_specs=pl.BlockSpec((1, T, D), lambda bb: (bb, 0, 0)),
        out_shape=jax.ShapeDtypeStruct((B, T, D), jnp.float32),
    )(g2, pos_embed, W, b2d)


def kernel(x, byte_embed, pos_embed, W, b):
    x_flat = x.reshape(ROWS).astype(jnp.int32)
    g2 = _sc_gather(x_flat, byte_embed)
    return _tc_finish(g2, pos_embed, W, b.reshape(1, D))


# speedup vs baseline: 1.1021x; 1.0997x over previous
import functools

import jax
import jax.numpy as jnp
from jax import lax
from jax.experimental import pallas as pl
from jax.experimental.pallas import tpu as pltpu
from jax.experimental.pallas import tpu_sc as plsc

D = 64
DP = 128
T = 4096
TH = T // 2
B = 4
V = 256
ROWS = B * T
QROWS = B * TH // 2
NC, NS, L = 2, 16, 16
NW = NC * NS
PPW = QROWS // NW


_MESH = plsc.VectorSubcoreMesh(core_axis_name="c", subcore_axis_name="s")


@functools.partial(
    pl.kernel,
    out_type=jax.ShapeDtypeStruct((QROWS, DP), jnp.float32),
    mesh=_MESH,
    compiler_params=pltpu.CompilerParams(use_tc_tiling_on_sc=False),
    scratch_types=[
        pltpu.VMEM((PPW,), jnp.int32),
        pltpu.VMEM((PPW,), jnp.int32),
        pltpu.VMEM((PPW, D), jnp.float32),
        pltpu.VMEM((PPW, D), jnp.float32),
        pltpu.SemaphoreType.DMA,
        pltpu.SemaphoreType.DMA,
    ],
)
def _sc_gather(x_hbm, table_hbm, out_hbm, idxa_v, idxb_v, rowsa_v, rowsb_v,
               sema, semb):
    wid = lax.axis_index("s") * NC + lax.axis_index("c")
    base = wid * PPW
    b01 = wid // 16
    flata = b01 * T + (wid % 16) * PPW
    pltpu.sync_copy(x_hbm.at[pl.ds(flata, PPW)], idxa_v)
    ga = pltpu.async_copy(table_hbm.at[idxa_v], rowsa_v, sema)
    pltpu.sync_copy(x_hbm.at[pl.ds(flata + 2 * T, PPW)], idxb_v)
    gb = pltpu.async_copy(table_hbm.at[idxb_v], rowsb_v, semb)
    out_slice = out_hbm.at[pl.ds(base, PPW)]
    ga.wait()
    pltpu.sync_copy(rowsa_v, out_slice.at[:, pl.ds(0, D)])
    gb.wait()
    pltpu.sync_copy(rowsb_v, out_slice.at[:, pl.ds(D, D)])


def _finish_body(g_ref, x_ref, byte_ref, pos_ref, w_ref, b_ref, y_ref):
    bb = pl.program_id(0)
    dn = (((1,), (1,)), ((), ()))
    w = w_ref[...]
    bias = b_ref[...]
    pos_lo = pos_ref[:TH]
    pos_hi = pos_ref[TH:]

    @pl.when(bb < 2)
    def _():
        y_ref[0, :TH] = lax.dot_general(
            g_ref[:, :D] + pos_lo, w, dn,
            preferred_element_type=jnp.float32) + bias

    @pl.when(bb >= 2)
    def _():
        y_ref[0, :TH] = lax.dot_general(
            g_ref[:, D:] + pos_lo, w, dn,
            preferred_element_type=jnp.float32) + bias

    idx = x_ref[0, 0]
    iota = lax.broadcasted_iota(jnp.int32, (TH, V), 1)
    oh = (idx[:, None] == iota).astype(jnp.float32)
    ge = lax.dot_general(oh, byte_ref[...], (((1,), (0,)), ((), ())),
                         preferred_element_type=jnp.float32)
    y_ref[0, TH:] = lax.dot_general(
        ge + pos_hi, w, dn, preferred_element_type=jnp.float32) + bias


def _tc_finish(g2, xhi3, byte_embed, pos_embed, W, b2d):
    return pl.pallas_call(
        _finish_body,
        grid=(B,),
        in_specs=[
            pl.BlockSpec((TH, DP), lambda bb: (bb % 2, 0)),
            pl.BlockSpec((1, 1, TH), lambda bb: (bb, 0, 0)),
            pl.BlockSpec((V, D), lambda bb: (0, 0)),
            pl.BlockSpec((T, D), lambda bb: (0, 0)),
            pl.BlockSpec((D, D), lambda bb: (0, 0)),
            pl.BlockSpec((1, D), lambda bb: (0, 0)),
        ],
        out_specs=pl.BlockSpec((1, T, D), lambda bb: (bb, 0, 0)),
        out_shape=jax.ShapeDtypeStruct((B, T, D), jnp.float32),
    )(g2, xhi3, byte_embed, pos_embed, W, b2d)


def kernel(x, byte_embed, pos_embed, W, b):
    xi = x.astype(jnp.int32)
    x_flat = xi.reshape(ROWS)
    g2 = _sc_gather(x_flat, byte_embed)
    xhi3 = xi[:, TH:].reshape(B, 1, TH)
    return _tc_finish(g2, xhi3, byte_embed, pos_embed, W, b.reshape(1, D))
